# baseline (speedup 1.0000x reference)
import functools

import jax
import jax.numpy as jnp
import numpy as np
from jax import lax
from jax.experimental import pallas as pl
from jax.experimental.pallas import tpu as pltpu
from jax.experimental.pallas import tpu_sc as plsc

BATCH = 16384
NUM_FIELDS = 19
EMBED_DIM = 16
CONCAT = NUM_FIELDS * EMBED_DIM
_OFFSETS = np.arange(NUM_FIELDS, dtype=np.int32) * 100000


_IDX_W = 128


def _sc_gather(table, idx3d):
    nw, rows_per_w, _ = idx3d.shape
    info = plsc.get_sparse_core_info()
    mesh = plsc.VectorSubcoreMesh(core_axis_name="c", subcore_axis_name="s")

    @functools.partial(
        pl.kernel,
        out_type=jax.ShapeDtypeStruct((nw * rows_per_w * _IDX_W, EMBED_DIM),
                                      jnp.float32),
        mesh=mesh,
        scratch_types=[
            pltpu.VMEM((rows_per_w, _IDX_W), jnp.int32),
            pltpu.VMEM((_IDX_W, EMBED_DIM), jnp.float32),
            pltpu.SemaphoreType.DMA,
        ],
        compiler_params=pltpu.CompilerParams(use_tc_tiling_on_sc=False),
    )
    def k(table_hbm, idx_hbm, out_hbm, idx_v, rows_v, sem):
        wid = lax.axis_index("s") * info.num_cores + lax.axis_index("c")
        base = wid * rows_per_w
        pltpu.sync_copy(idx_hbm.at[wid], idx_v)

        def body(j, carry):
            pltpu.async_copy(table_hbm.at[idx_v.at[j]], rows_v, sem).wait()
            pltpu.sync_copy(rows_v, out_hbm.at[pl.ds((base + j) * _IDX_W, _IDX_W)])
            return carry

        lax.fori_loop(0, rows_per_w, body, 0)

    return k(table, idx3d)


_BLK = 1024


def _tc_body(e_ref, w1_ref, b1_ref, w2_ref, b2_ref, w3_ref, b3_ref,
             wo_ref, cb_ref, out_ref):
    e = e_ref[...]
    r = lax.broadcasted_iota(jnp.int32, (CONCAT, EMBED_DIM), 0)
    c = lax.broadcasted_iota(jnp.int32, (CONCAT, EMBED_DIM), 1)
    m = (r % EMBED_DIM == c).astype(jnp.float32)
    t = jnp.dot(e, m, preferred_element_type=jnp.float32)
    u = jnp.dot(e * e, m, preferred_element_type=jnp.float32)
    linear = jnp.sum(t, axis=1)
    fm = 0.5 * (jnp.sum(t * t, axis=1) - jnp.sum(u, axis=1))
    h = jnp.maximum(jnp.dot(e, w1_ref[...], preferred_element_type=jnp.float32)
                    + b1_ref[...], 0.0)
    h = jnp.maximum(jnp.dot(h, w2_ref[...], preferred_element_type=jnp.float32)
                    + b2_ref[...], 0.0)
    h = jnp.maximum(jnp.dot(h, w3_ref[...], preferred_element_type=jnp.float32)
                    + b3_ref[...], 0.0)
    mlp = jnp.sum(h * wo_ref[...], axis=1)
    logit = linear + fm + mlp + cb_ref[0, 0]
    out_ref[...] = jax.nn.sigmoid(logit)[None, None, :]


def _tc_forward(e, w1, b1, w2, b2, w3, b3, wo, cb):
    grid = BATCH // _BLK
    full = lambda i: (0, 0)
    out = pl.pallas_call(
        _tc_body,
        grid=(grid,),
        in_specs=[
            pl.BlockSpec((_BLK, CONCAT), lambda i: (i, 0)),
            pl.BlockSpec(w1.shape, full),
            pl.BlockSpec(b1.shape, full),
            pl.BlockSpec(w2.shape, full),
            pl.BlockSpec(b2.shape, full),
            pl.BlockSpec(w3.shape, full),
            pl.BlockSpec(b3.shape, full),
            pl.BlockSpec(wo.shape, full),
            pl.BlockSpec(cb.shape, full),
        ],
        out_specs=pl.BlockSpec((1, 1, _BLK), lambda i: (i, 0, 0)),
        out_shape=jax.ShapeDtypeStruct((grid, 1, _BLK), jnp.float32),
    )(e, w1, b1, w2, b2, w3, b3, wo, cb)
    return out.reshape(BATCH)


def kernel(x, additional, column, table, bias, W1, b1, g1, be1, W2, b2, g2,
           be2, W3, b3, g3, be3, Wo, bo):
    x_sel = jnp.concatenate([x[:, 0:1], x[:, 5:6], x[:, 17:21], x[:, 26:]],
                            axis=1)
    idx = (x_sel + jnp.asarray(_OFFSETS)).astype(jnp.int32)
    idx3d = idx.reshape(32, -1, _IDX_W)

    embed_flat = _sc_gather(table, idx3d)
    e = embed_flat.reshape(BATCH, CONCAT)

    inv = np.float32(1.0 / np.sqrt(1.0 + 1e-5))
    s1, s2, s3 = inv * g1, inv * g2, inv * g3
    w1 = W1 * s1[None, :]
    w2 = W2 * s2[None, :]
    w3 = W3 * s3[None, :]
    b1e = (b1 * s1 + be1)[None, :]
    b2e = (b2 * s2 + be2)[None, :]
    b3e = (b3 * s3 + be3)[None, :]
    wo = Wo.reshape(1, -1)
    cb = (bias[0] + bo[0]).reshape(1, 1)

    return _tc_forward(e, w1, b1e, w2, b2e, w3, b3e, wo, cb)


# speedup vs baseline: 1.0612x; 1.0612x over previous
import functools

import jax
import jax.numpy as jnp
import numpy as np
from jax import lax
from jax.experimental import pallas as pl
from jax.experimental.pallas import tpu as pltpu
from jax.experimental.pallas import tpu_sc as plsc

BATCH = 16384
NUM_FIELDS = 19
EMBED_DIM = 16
CONCAT = NUM_FIELDS * EMBED_DIM
NPLANE = 3
TABLE_ROWS = 1900000
_OFFSETS = np.arange(NUM_FIELDS, dtype=np.int32) * 100000

_IDX_W = 128


def _sc_gather_scatter(table_lin, idx3d, g3d):
    nw, rows_per_w, _ = idx3d.shape
    info = plsc.get_sparse_core_info()
    mesh = plsc.VectorSubcoreMesh(core_axis_name="c", subcore_axis_name="s")
    out_rows = NPLANE * BATCH * 8

    @functools.partial(
        pl.kernel,
        out_type=jax.ShapeDtypeStruct((out_rows, EMBED_DIM), jnp.float32),
        mesh=mesh,
        scratch_types=[
            pltpu.VMEM((rows_per_w, _IDX_W), jnp.int32),
            pltpu.VMEM((rows_per_w, _IDX_W), jnp.int32),
            pltpu.VMEM((2, _IDX_W, EMBED_DIM), jnp.float32),
            pltpu.SemaphoreType.DMA,
            pltpu.SemaphoreType.DMA,
        ],
        compiler_params=pltpu.CompilerParams(use_tc_tiling_on_sc=False),
    )
    def k(table_hbm, idx_hbm, g_hbm, out_hbm, idx_v, g_v, rows_v, gsem, ssem):
        wid = lax.axis_index("s") * info.num_cores + lax.axis_index("c")
        pltpu.sync_copy(idx_hbm.at[wid], idx_v)
        pltpu.sync_copy(g_hbm.at[wid], g_v)

        pltpu.async_copy(table_hbm.at[idx_v.at[0]], rows_v.at[0], gsem)

        def body(j, carry):
            slot = lax.rem(j, 2)
            nslot = lax.rem(j + 1, 2)

            @pl.when(j > 0)
            def _():
                pltpu.make_async_copy(rows_v.at[nslot],
                                      out_hbm.at[g_v.at[j]], ssem).wait()

            @pl.when(j + 1 < rows_per_w)
            def _():
                pltpu.async_copy(table_hbm.at[idx_v.at[j + 1]],
                                 rows_v.at[nslot], gsem)

            pltpu.make_async_copy(table_hbm.at[idx_v.at[0]],
                                  rows_v.at[slot], gsem).wait()
            pltpu.async_copy(rows_v.at[slot], out_hbm.at[g_v.at[j]], ssem)
            return carry

        lax.fori_loop(0, rows_per_w, body, 0)
        pltpu.make_async_copy(rows_v.at[0], out_hbm.at[g_v.at[0]], ssem).wait()

    return k(table_lin, idx3d, g3d)


_BLK = 1024


def _tc_body(e_ref, w1a_ref, w1b_ref, w1c_ref, b1_ref, w2_ref, b2_ref,
             w3_ref, b3_ref, wo_ref, cb_ref, out_ref):
    x0 = e_ref[0]
    x1 = e_ref[1]
    x2 = e_ref[2]
    lane = lax.broadcasted_iota(jnp.int32, (_BLK, 128), 1)
    x2 = jnp.where(lane < CONCAT - 256, x2, 0.0)
    r = lax.broadcasted_iota(jnp.int32, (128, EMBED_DIM), 0)
    c = lax.broadcasted_iota(jnp.int32, (128, EMBED_DIM), 1)
    m = (r % EMBED_DIM == c).astype(jnp.float32)
    xs = x0 + x1 + x2
    xq = x0 * x0 + x1 * x1 + x2 * x2
    t = jnp.dot(xs, m, preferred_element_type=jnp.float32)
    u = jnp.dot(xq, m, preferred_element_type=jnp.float32)
    linear = jnp.sum(t, axis=1)
    fm = 0.5 * (jnp.sum(t * t, axis=1) - jnp.sum(u, axis=1))
    h = jnp.dot(x0, w1a_ref[...], preferred_element_type=jnp.float32)
    h += jnp.dot(x1, w1b_ref[...], preferred_element_type=jnp.float32)
    h += jnp.dot(x2, w1c_ref[...], preferred_element_type=jnp.float32)
    h = jnp.maximum(h + b1_ref[...], 0.0)
    h = jnp.maximum(jnp.dot(h, w2_ref[...], preferred_element_type=jnp.float32)
                    + b2_ref[...], 0.0)
    h = jnp.maximum(jnp.dot(h, w3_ref[...], preferred_element_type=jnp.float32)
                    + b3_ref[...], 0.0)
    mlp = jnp.sum(h * wo_ref[...], axis=1)
    logit = linear + fm + mlp + cb_ref[0, 0]
    out_ref[...] = jax.nn.sigmoid(logit)[None, None, :]


def _tc_forward(e3, w1a, w1b, w1c, b1, w2, b2, w3, b3, wo, cb):
    grid = BATCH // _BLK
    full = lambda i: (0, 0)
    out = pl.pallas_call(
        _tc_body,
        grid=(grid,),
        in_specs=[
            pl.BlockSpec((NPLANE, _BLK, 128), lambda i: (0, i, 0)),
            pl.BlockSpec(w1a.shape, full),
            pl.BlockSpec(w1b.shape, full),
            pl.BlockSpec(w1c.shape, full),
            pl.BlockSpec(b1.shape, full),
            pl.BlockSpec(w2.shape, full),
            pl.BlockSpec(b2.shape, full),
            pl.BlockSpec(w3.shape, full),
            pl.BlockSpec(b3.shape, full),
            pl.BlockSpec(wo.shape, full),
            pl.BlockSpec(cb.shape, full),
        ],
        out_specs=pl.BlockSpec((1, 1, _BLK), lambda i: (i, 0, 0)),
        out_shape=jax.ShapeDtypeStruct((grid, 1, _BLK), jnp.float32),
    )(e3, w1a, w1b, w1c, b1, w2, b2, w3, b3, wo, cb)
    return out.reshape(BATCH)


def kernel(x, additional, column, table, bias, W1, b1, g1, be1, W2, b2, g2,
           be2, W3, b3, g3, be3, Wo, bo):
    x_sel = jnp.concatenate([x[:, 0:1], x[:, 5:6], x[:, 17:21], x[:, 26:]],
                            axis=1)
    idx = (x_sel + jnp.asarray(_OFFSETS)).astype(jnp.int32)
    idx3d = idx.reshape(32, -1, _IDX_W)
    b_row = jnp.arange(BATCH, dtype=jnp.int32)[:, None]
    f_col = jnp.arange(NUM_FIELDS, dtype=jnp.int32)[None, :]
    g = (f_col // 8) * (BATCH * 8) + b_row * 8 + (f_col % 8)
    g3d = g.reshape(32, -1, _IDX_W)

    table_lin = lax.optimization_barrier(table.reshape(-1))
    table_lin = table_lin.reshape(TABLE_ROWS, EMBED_DIM)

    granules = _sc_gather_scatter(table_lin, idx3d, g3d)
    e3 = granules.reshape(NPLANE, BATCH, 128)

    inv = np.float32(1.0 / np.sqrt(1.0 + 1e-5))
    s1, s2, s3 = inv * g1, inv * g2, inv * g3
    w1 = W1 * s1[None, :]
    w1a = w1[0:128]
    w1b = w1[128:256]
    w1c = jnp.pad(w1[256:304], ((0, 80), (0, 0)))
    w2 = W2 * s2[None, :]
    w3 = W3 * s3[None, :]
    b1e = (b1 * s1 + be1)[None, :]
    b2e = (b2 * s2 + be2)[None, :]
    b3e = (b3 * s3 + be3)[None, :]
    wo = Wo.reshape(1, -1)
    cb = (bias[0] + bo[0]).reshape(1, 1)

    return _tc_forward(e3, w1a, w1b, w1c, b1e, w2, b2e, w3, b3e, wo, cb)


# speedup vs baseline: 3.1195x; 2.9396x over previous
import functools

import jax
import jax.numpy as jnp
import numpy as np
from jax import lax
from jax.experimental import pallas as pl
from jax.experimental.pallas import tpu as pltpu
from jax.experimental.pallas import tpu_sc as plsc

BATCH = 16384
NUM_FIELDS = 19
EMBED_DIM = 16
CONCAT = NUM_FIELDS * EMBED_DIM
NPLANE = 3
TABLE_ROWS = 1900000
_OFFSETS = np.arange(NUM_FIELDS, dtype=np.int32) * 100000
_FIELD_COLS = np.array([0, 5, 17, 18, 19, 20] + list(range(26, 39)))

_IDX_W = 128
_NW = 32

_TB_COLS = 16384
_TB_GRID = 116
_PAD_ROWS = _TB_GRID * _TB_COLS

_FF = np.arange(NUM_FIELDS, dtype=np.int64)[:, None]
_BB = np.arange(BATCH, dtype=np.int64)[None, :]
_G3D = ((_FF // 8) * (BATCH * 8) + _BB * 8 + (_FF % 8)).astype(np.int32)
_G3D = _G3D.reshape(_NW, -1, _IDX_W)


def _transpose_body(t_ref, out_ref):
    x = t_ref[...]
    a = x.reshape(16, 128, 128).reshape(2048, 128)
    b = jnp.transpose(a)
    c = b.reshape(128, 16, 128).reshape(2048, 128)
    d = jnp.transpose(c)
    out_ref[...] = d.reshape(128, 16, 128).reshape(2048, 128)


def _relayout_table(table_t):
    out = pl.pallas_call(
        _transpose_body,
        grid=(_TB_GRID,),
        in_specs=[pl.BlockSpec((EMBED_DIM, _TB_COLS), lambda i: (0, i))],
        out_specs=pl.BlockSpec((_TB_COLS // 8, 128), lambda i: (i, 0)),
        out_shape=jax.ShapeDtypeStruct((_PAD_ROWS // 8, 128), jnp.float32),
    )(table_t)
    return out.reshape(-1).reshape(_PAD_ROWS, EMBED_DIM)


def _sc_gather_scatter(table_lin, idx3d, g3d):
    nw, rows_per_w, _ = idx3d.shape
    info = plsc.get_sparse_core_info()
    mesh = plsc.VectorSubcoreMesh(core_axis_name="c", subcore_axis_name="s")
    out_rows = NPLANE * BATCH * 8

    @functools.partial(
        pl.kernel,
        out_type=jax.ShapeDtypeStruct((out_rows, EMBED_DIM), jnp.float32),
        mesh=mesh,
        scratch_types=[
            pltpu.VMEM((rows_per_w, _IDX_W), jnp.int32),
            pltpu.VMEM((rows_per_w, _IDX_W), jnp.int32),
            pltpu.VMEM((2, _IDX_W, EMBED_DIM), jnp.float32),
            pltpu.SemaphoreType.DMA,
            pltpu.SemaphoreType.DMA,
        ],
        compiler_params=pltpu.CompilerParams(use_tc_tiling_on_sc=False),
    )
    def k(table_hbm, idx_hbm, g_hbm, out_hbm, idx_v, g_v, rows_v, gsem, ssem):
        wid = lax.axis_index("s") * info.num_cores + lax.axis_index("c")
        pltpu.sync_copy(idx_hbm.at[wid], idx_v)
        pltpu.sync_copy(g_hbm.at[wid], g_v)

        pltpu.async_copy(table_hbm.at[idx_v.at[0]], rows_v.at[0], gsem)

        def body(j, carry):
            slot = lax.rem(j, 2)
            nslot = lax.rem(j + 1, 2)

            @pl.when(j > 0)
            def _():
                pltpu.make_async_copy(rows_v.at[nslot],
                                      out_hbm.at[g_v.at[j]], ssem).wait()

            @pl.when(j + 1 < rows_per_w)
            def _():
                pltpu.async_copy(table_hbm.at[idx_v.at[j + 1]],
                                 rows_v.at[nslot], gsem)

            pltpu.make_async_copy(table_hbm.at[idx_v.at[0]],
                                  rows_v.at[slot], gsem).wait()
            pltpu.async_copy(rows_v.at[slot], out_hbm.at[g_v.at[j]], ssem)
            return carry

        lax.fori_loop(0, rows_per_w, body, 0)
        pltpu.make_async_copy(rows_v.at[0], out_hbm.at[g_v.at[0]], ssem).wait()

    return k(table_lin, idx3d, g3d)


_BLK = 1024


def _tc_body(e_ref, w1a_ref, w1b_ref, w1c_ref, b1_ref, w2_ref, b2_ref,
             w3_ref, b3_ref, wo_ref, cb_ref, out_ref):
    x0 = e_ref[0]
    x1 = e_ref[1]
    x2 = e_ref[2]
    lane = lax.broadcasted_iota(jnp.int32, (_BLK, 128), 1)
    x2 = jnp.where(lane < CONCAT - 256, x2, 0.0)
    r = lax.broadcasted_iota(jnp.int32, (128, EMBED_DIM), 0)
    c = lax.broadcasted_iota(jnp.int32, (128, EMBED_DIM), 1)
    m = (r % EMBED_DIM == c).astype(jnp.float32)
    xs = x0 + x1 + x2
    xq = x0 * x0 + x1 * x1 + x2 * x2
    t = jnp.dot(xs, m, preferred_element_type=jnp.float32)
    u = jnp.dot(xq, m, preferred_element_type=jnp.float32)
    linear = jnp.sum(t, axis=1)
    fm = 0.5 * (jnp.sum(t * t, axis=1) - jnp.sum(u, axis=1))
    h = jnp.dot(x0, w1a_ref[...], preferred_element_type=jnp.float32)
    h += jnp.dot(x1, w1b_ref[...], preferred_element_type=jnp.float32)
    h += jnp.dot(x2, w1c_ref[...], preferred_element_type=jnp.float32)
    h = jnp.maximum(h + b1_ref[...], 0.0)
    h = jnp.maximum(jnp.dot(h, w2_ref[...], preferred_element_type=jnp.float32)
                    + b2_ref[...], 0.0)
    h = jnp.maximum(jnp.dot(h, w3_ref[...], preferred_element_type=jnp.float32)
                    + b3_ref[...], 0.0)
    mlp = jnp.sum(h * wo_ref[...], axis=1)
    logit = linear + fm + mlp + cb_ref[0, 0]
    out_ref[...] = jax.nn.sigmoid(logit)[None, None, :]


def _tc_forward(e3, w1a, w1b, w1c, b1, w2, b2, w3, b3, wo, cb):
    grid = BATCH // _BLK
    full = lambda i: (0, 0)
    out = pl.pallas_call(
        _tc_body,
        grid=(grid,),
        in_specs=[
            pl.BlockSpec((NPLANE, _BLK, 128), lambda i: (0, i, 0)),
            pl.BlockSpec(w1a.shape, full),
            pl.BlockSpec(w1b.shape, full),
            pl.BlockSpec(w1c.shape, full),
            pl.BlockSpec(b1.shape, full),
            pl.BlockSpec(w2.shape, full),
            pl.BlockSpec(b2.shape, full),
            pl.BlockSpec(w3.shape, full),
            pl.BlockSpec(b3.shape, full),
            pl.BlockSpec(wo.shape, full),
            pl.BlockSpec(cb.shape, full),
        ],
        out_specs=pl.BlockSpec((1, 1, _BLK), lambda i: (i, 0, 0)),
        out_shape=jax.ShapeDtypeStruct((grid, 1, _BLK), jnp.float32),
    )(e3, w1a, w1b, w1c, b1, w2, b2, w3, b3, wo, cb)
    return out.reshape(BATCH)


def kernel(x, additional, column, table, bias, W1, b1, g1, be1, W2, b2, g2,
           be2, W3, b3, g3, be3, Wo, bo):
    xt = x.T
    x_sel = xt[jnp.asarray(_FIELD_COLS)]
    idx = x_sel + jnp.asarray(_OFFSETS)[:, None]
    idx3d = idx.astype(jnp.int32).reshape(_NW, -1, _IDX_W)
    g3d = jnp.asarray(_G3D)

    table_lin = _relayout_table(table.T)

    granules = _sc_gather_scatter(table_lin, idx3d, g3d)
    e3 = granules.reshape(NPLANE, BATCH, 128)

    inv = np.float32(1.0 / np.sqrt(1.0 + 1e-5))
    s1, s2, s3 = inv * g1, inv * g2, inv * g3
    w1 = W1 * s1[None, :]
    w1a = w1[0:128]
    w1b = w1[128:256]
    w1c = jnp.pad(w1[256:304], ((0, 80), (0, 0)))
    w2 = W2 * s2[None, :]
    w3 = W3 * s3[None, :]
    b1e = (b1 * s1 + be1)[None, :]
    b2e = (b2 * s2 + be2)[None, :]
    b3e = (b3 * s3 + be3)[None, :]
    wo = Wo.reshape(1, -1)
    cb = (bias[0] + bo[0]).reshape(1, 1)

    return _tc_forward(e3, w1a, w1b, w1c, b1e, w2, b2e, w3, b3e, wo, cb)


# speedup vs baseline: 3.4661x; 1.1111x over previous
import functools

import jax
import jax.numpy as jnp
import numpy as np
from jax import lax
from jax.experimental import pallas as pl
from jax.experimental.pallas import tpu as pltpu
from jax.experimental.pallas import tpu_sc as plsc

BATCH = 16384
NUM_FIELDS = 19
EMBED_DIM = 16
CONCAT = NUM_FIELDS * EMBED_DIM
TABLE_ROWS = 1900000
_OFFSETS = np.arange(NUM_FIELDS, dtype=np.int32) * 100000
_FIELD_COLS = np.array([0, 5, 17, 18, 19, 20] + list(range(26, 39)))

_IDX_W = 128
_NW = 32
_BH = BATCH // 2

_TB_COLS = 16384
_TB_GRID = 116
_A_BLKS = 62
_B_LO = 61
_B_BLKS = _TB_GRID - _B_LO
_B_BASE = _B_LO * _TB_COLS

_FA = np.arange(10, dtype=np.int64)[:, None]
_BA = np.arange(BATCH, dtype=np.int64)[None, :]
_GA = ((_FA // 8) * (BATCH * 8) + _BA * 8 + (_FA % 8)).astype(np.int32)
_GA = _GA.reshape(_NW, -1, _IDX_W)
_FB = np.arange(10, 19, dtype=np.int64)[:, None]
_BB = np.arange(_BH, dtype=np.int64)[None, :]
_GB = (((_FB // 8) - 1) * (_BH * 8) + _BB * 8 + (_FB % 8)).astype(np.int32)
_GB = _GB.reshape(_NW, -1, _IDX_W)


def _transpose_body(t_ref, out_ref):
    x = t_ref[...]
    a = x.reshape(16, 128, 128).reshape(2048, 128)
    b = jnp.transpose(a)
    c = b.reshape(128, 16, 128).reshape(2048, 128)
    d = jnp.transpose(c)
    out_ref[...] = d.reshape(128, 16, 128).reshape(2048, 128)


def _relayout_table(table_t, blk_lo, nblk):
    rows = nblk * _TB_COLS
    out = pl.pallas_call(
        _transpose_body,
        grid=(nblk,),
        in_specs=[pl.BlockSpec((EMBED_DIM, _TB_COLS),
                               lambda i: (0, blk_lo + i))],
        out_specs=pl.BlockSpec((_TB_COLS // 8, 128), lambda i: (i, 0)),
        out_shape=jax.ShapeDtypeStruct((rows // 8, 128), jnp.float32),
    )(table_t)
    return out.reshape(-1).reshape(rows, EMBED_DIM)


def _sc_gather_scatter(table_lin, idx3d, g3d, nplanes, nbatch):
    nw, rows_per_w, _ = idx3d.shape
    info = plsc.get_sparse_core_info()
    mesh = plsc.VectorSubcoreMesh(core_axis_name="c", subcore_axis_name="s")
    out_rows = nplanes * nbatch * 8

    @functools.partial(
        pl.kernel,
        out_type=jax.ShapeDtypeStruct((out_rows, EMBED_DIM), jnp.float32),
        mesh=mesh,
        scratch_types=[
            pltpu.VMEM((rows_per_w, _IDX_W), jnp.int32),
            pltpu.VMEM((rows_per_w, _IDX_W), jnp.int32),
            pltpu.VMEM((2, _IDX_W, EMBED_DIM), jnp.float32),
            pltpu.SemaphoreType.DMA,
            pltpu.SemaphoreType.DMA,
        ],
        compiler_params=pltpu.CompilerParams(use_tc_tiling_on_sc=False),
    )
    def k(table_hbm, idx_hbm, g_hbm, out_hbm, idx_v, g_v, rows_v, gsem, ssem):
        wid = lax.axis_index("s") * info.num_cores + lax.axis_index("c")
        pltpu.sync_copy(idx_hbm.at[wid], idx_v)
        pltpu.sync_copy(g_hbm.at[wid], g_v)

        pltpu.async_copy(table_hbm.at[idx_v.at[0]], rows_v.at[0], gsem)

        def body(j, carry):
            slot = lax.rem(j, 2)
            nslot = lax.rem(j + 1, 2)

            @pl.when(j > 0)
            def _():
                pltpu.make_async_copy(rows_v.at[nslot],
                                      out_hbm.at[g_v.at[j]], ssem).wait()

            @pl.when(j + 1 < rows_per_w)
            def _():
                pltpu.async_copy(table_hbm.at[idx_v.at[j + 1]],
                                 rows_v.at[nslot], gsem)

            pltpu.make_async_copy(table_hbm.at[idx_v.at[0]],
                                  rows_v.at[slot], gsem).wait()
            pltpu.async_copy(rows_v.at[slot], out_hbm.at[g_v.at[j]], ssem)
            return carry

        lax.fori_loop(0, rows_per_w, body, 0)
        pltpu.make_async_copy(rows_v.at[0], out_hbm.at[g_v.at[0]], ssem).wait()

    return k(table_lin, idx3d, g3d)


_BLK = 1024


def _tc_body(a_ref, b_ref, w1a_ref, w1b_ref, w1c_ref, b1_ref, w2_ref, b2_ref,
             w3_ref, b3_ref, wo_ref, cb_ref, out_ref):
    lane = lax.broadcasted_iota(jnp.int32, (_BLK, 128), 1)
    x0 = a_ref[0]
    x1 = jnp.where(lane < 32, a_ref[1], b_ref[0])
    x2 = jnp.where(lane < CONCAT - 256, b_ref[1], 0.0)
    r = lax.broadcasted_iota(jnp.int32, (128, EMBED_DIM), 0)
    c = lax.broadcasted_iota(jnp.int32, (128, EMBED_DIM), 1)
    m = (r % EMBED_DIM == c).astype(jnp.float32)
    xs = x0 + x1 + x2
    xq = x0 * x0 + x1 * x1 + x2 * x2
    t = jnp.dot(xs, m, preferred_element_type=jnp.float32)
    u = jnp.dot(xq, m, preferred_element_type=jnp.float32)
    linear = jnp.sum(t, axis=1)
    fm = 0.5 * (jnp.sum(t * t, axis=1) - jnp.sum(u, axis=1))
    h = jnp.dot(x0, w1a_ref[...], preferred_element_type=jnp.float32)
    h += jnp.dot(x1, w1b_ref[...], preferred_element_type=jnp.float32)
    h += jnp.dot(x2, w1c_ref[...], preferred_element_type=jnp.float32)
    h = jnp.maximum(h + b1_ref[...], 0.0)
    h = jnp.maximum(jnp.dot(h, w2_ref[...], preferred_element_type=jnp.float32)
                    + b2_ref[...], 0.0)
    h = jnp.maximum(jnp.dot(h, w3_ref[...], preferred_element_type=jnp.float32)
                    + b3_ref[...], 0.0)
    mlp = jnp.sum(h * wo_ref[...], axis=1)
    logit = linear + fm + mlp + cb_ref[0, 0]
    out_ref[...] = jax.nn.sigmoid(logit)[None, None, :]


def _tc_forward(h, eA, eB, w1a, w1b, w1c, b1, w2, b2, w3, b3, wo, cb):
    grid = _BH // _BLK
    full = lambda i: (0, 0)
    out = pl.pallas_call(
        _tc_body,
        grid=(grid,),
        in_specs=[
            pl.BlockSpec((2, _BLK, 128), lambda i: (0, h * (_BH // _BLK) + i, 0)),
            pl.BlockSpec((2, _BLK, 128), lambda i: (0, i, 0)),
            pl.BlockSpec(w1a.shape, full),
            pl.BlockSpec(w1b.shape, full),
            pl.BlockSpec(w1c.shape, full),
            pl.BlockSpec(b1.shape, full),
            pl.BlockSpec(w2.shape, full),
            pl.BlockSpec(b2.shape, full),
            pl.BlockSpec(w3.shape, full),
            pl.BlockSpec(b3.shape, full),
            pl.BlockSpec(wo.shape, full),
            pl.BlockSpec(cb.shape, full),
        ],
        out_specs=pl.BlockSpec((1, 1, _BLK), lambda i: (i, 0, 0)),
        out_shape=jax.ShapeDtypeStruct((grid, 1, _BLK), jnp.float32),
    )(eA, eB, w1a, w1b, w1c, b1, w2, b2, w3, b3, wo, cb)
    return out.reshape(_BH)


def kernel(x, additional, column, table, bias, W1, b1, g1, be1, W2, b2, g2,
           be2, W3, b3, g3, be3, Wo, bo):
    xt = x.T
    x_sel = xt[jnp.asarray(_FIELD_COLS)]
    idx = (x_sel + jnp.asarray(_OFFSETS)[:, None]).astype(jnp.int32)
    idxA = idx[:10].reshape(_NW, -1, _IDX_W)
    gA = jnp.asarray(_GA)
    gB = jnp.asarray(_GB)

    table_t = table.T
    tlA = _relayout_table(table_t, 0, _A_BLKS)
    tlB = _relayout_table(table_t, _B_LO, _B_BLKS)

    outA = _sc_gather_scatter(tlA, idxA, gA, 2, BATCH)
    eA = outA.reshape(2, BATCH, 128)
    eBs = []
    for h in range(2):
        idxB = (idx[10:, h * _BH:(h + 1) * _BH] - _B_BASE
                ).reshape(_NW, -1, _IDX_W)
        outB = _sc_gather_scatter(tlB, idxB, gB, 2, _BH)
        eBs.append(outB.reshape(2, _BH, 128))

    inv = np.float32(1.0 / np.sqrt(1.0 + 1e-5))
    s1, s2, s3 = inv * g1, inv * g2, inv * g3
    w1 = W1 * s1[None, :]
    w1a = w1[0:128]
    w1b = w1[128:256]
    w1c = jnp.pad(w1[256:304], ((0, 80), (0, 0)))
    w2 = W2 * s2[None, :]
    w3 = W3 * s3[None, :]
    b1e = (b1 * s1 + be1)[None, :]
    b2e = (b2 * s2 + be2)[None, :]
    b3e = (b3 * s3 + be3)[None, :]
    wo = Wo.reshape(1, -1)
    cb = (bias[0] + bo[0]).reshape(1, 1)

    outs = [_tc_forward(h, eA, eBs[h], w1a, w1b, w1c, b1e, w2, b2e, w3, b3e,
                        wo, cb)
            for h in range(2)]
    return jnp.concatenate(outs)


# speedup vs baseline: 3.4890x; 1.0066x over previous
import functools

import jax
import jax.numpy as jnp
import numpy as np
from jax import lax
from jax.experimental import pallas as pl
from jax.experimental.pallas import tpu as pltpu
from jax.experimental.pallas import tpu_sc as plsc

BATCH = 16384
NUM_FIELDS = 19
EMBED_DIM = 16
CONCAT = NUM_FIELDS * EMBED_DIM
TABLE_ROWS = 1900000
_OFFSETS = np.arange(NUM_FIELDS, dtype=np.int32) * 100000
_FIELD_COLS = np.array([0, 5, 17, 18, 19, 20] + list(range(26, 39)))

_IDX_W = 128
_NW = 32
_BH = BATCH // 2

_TB_COLS = 16384
_TB_GRID = 116
_A_BLKS = 62
_B_LO, _B_BLKS = 61, 37
_B_BASE = _B_LO * _TB_COLS
_C_LO, _C_BLKS = 97, _TB_GRID - 97
_C_BASE = _C_LO * _TB_COLS

_FA = np.arange(10, dtype=np.int64)[:, None]
_BA = np.arange(BATCH, dtype=np.int64)[None, :]
_GA = ((_FA // 8) * (BATCH * 8) + _BA * 8 + (_FA % 8)).astype(np.int32)
_GA = _GA.reshape(_NW, -1, _IDX_W)
_FB = np.arange(10, 16, dtype=np.int64)[:, None]
_GB = (_BA * 8 + (_FB % 8)).astype(np.int32)
_GB = _GB.reshape(_NW, -1, _IDX_W)
_FC = np.arange(16, 19, dtype=np.int64)[:, None]
_BC = np.arange(_BH, dtype=np.int64)[None, :]
_GC = (_BC * 8 + (_FC % 8)).astype(np.int32)
_GC = _GC.reshape(_NW, -1, _IDX_W)


def _transpose_body(t_ref, out_ref):
    x = t_ref[...]
    a = x.reshape(16, 128, 128).reshape(2048, 128)
    b = jnp.transpose(a)
    c = b.reshape(128, 16, 128).reshape(2048, 128)
    d = jnp.transpose(c)
    out_ref[...] = d.reshape(128, 16, 128).reshape(2048, 128)


def _relayout_table(table_t, blk_lo, nblk):
    rows = nblk * _TB_COLS
    out = pl.pallas_call(
        _transpose_body,
        grid=(nblk,),
        in_specs=[pl.BlockSpec((EMBED_DIM, _TB_COLS),
                               lambda i: (0, blk_lo + i))],
        out_specs=pl.BlockSpec((_TB_COLS // 8, 128), lambda i: (i, 0)),
        out_shape=jax.ShapeDtypeStruct((rows // 8, 128), jnp.float32),
    )(table_t)
    return out.reshape(-1).reshape(rows, EMBED_DIM)


def _sc_gather_scatter(table_lin, idx3d, g3d, nplanes, nbatch):
    nw, rows_per_w, _ = idx3d.shape
    info = plsc.get_sparse_core_info()
    mesh = plsc.VectorSubcoreMesh(core_axis_name="c", subcore_axis_name="s")
    out_rows = nplanes * nbatch * 8

    @functools.partial(
        pl.kernel,
        out_type=jax.ShapeDtypeStruct((out_rows, EMBED_DIM), jnp.float32),
        mesh=mesh,
        scratch_types=[
            pltpu.VMEM((rows_per_w, _IDX_W), jnp.int32),
            pltpu.VMEM((rows_per_w, _IDX_W), jnp.int32),
            pltpu.VMEM((2, _IDX_W, EMBED_DIM), jnp.float32),
            pltpu.SemaphoreType.DMA,
            pltpu.SemaphoreType.DMA,
        ],
        compiler_params=pltpu.CompilerParams(use_tc_tiling_on_sc=False),
    )
    def k(table_hbm, idx_hbm, g_hbm, out_hbm, idx_v, g_v, rows_v, gsem, ssem):
        wid = lax.axis_index("s") * info.num_cores + lax.axis_index("c")
        pltpu.sync_copy(idx_hbm.at[wid], idx_v)
        pltpu.sync_copy(g_hbm.at[wid], g_v)

        pltpu.async_copy(table_hbm.at[idx_v.at[0]], rows_v.at[0], gsem)

        def body(j, carry):
            slot = lax.rem(j, 2)
            nslot = lax.rem(j + 1, 2)

            @pl.when(j > 0)
            def _():
                pltpu.make_async_copy(rows_v.at[nslot],
                                      out_hbm.at[g_v.at[j]], ssem).wait()

            @pl.when(j + 1 < rows_per_w)
            def _():
                pltpu.async_copy(table_hbm.at[idx_v.at[j + 1]],
                                 rows_v.at[nslot], gsem)

            pltpu.make_async_copy(table_hbm.at[idx_v.at[0]],
                                  rows_v.at[slot], gsem).wait()
            pltpu.async_copy(rows_v.at[slot], out_hbm.at[g_v.at[j]], ssem)
            return carry

        lax.fori_loop(0, rows_per_w, body, 0)
        pltpu.make_async_copy(rows_v.at[0], out_hbm.at[g_v.at[0]], ssem).wait()

    return k(table_lin, idx3d, g3d)


_BLK = 1024


def _tc_body(a_ref, b_ref, c_ref, w1a_ref, w1b_ref, w1c_ref, b1_ref, w2_ref,
             b2_ref, w3_ref, b3_ref, wo_ref, cb_ref, out_ref):
    lane = lax.broadcasted_iota(jnp.int32, (_BLK, 128), 1)
    x0 = a_ref[0]
    x1 = jnp.where(lane < 32, a_ref[1], b_ref[0])
    x2 = jnp.where(lane < CONCAT - 256, c_ref[0], 0.0)
    r = lax.broadcasted_iota(jnp.int32, (128, EMBED_DIM), 0)
    c = lax.broadcasted_iota(jnp.int32, (128, EMBED_DIM), 1)
    m = (r % EMBED_DIM == c).astype(jnp.float32)
    xs = x0 + x1 + x2
    xq = x0 * x0 + x1 * x1 + x2 * x2
    t = jnp.dot(xs, m, preferred_element_type=jnp.float32)
    u = jnp.dot(xq, m, preferred_element_type=jnp.float32)
    linear = jnp.sum(t, axis=1)
    fm = 0.5 * (jnp.sum(t * t, axis=1) - jnp.sum(u, axis=1))
    h = jnp.dot(x0, w1a_ref[...], preferred_element_type=jnp.float32)
    h += jnp.dot(x1, w1b_ref[...], preferred_element_type=jnp.float32)
    h += jnp.dot(x2, w1c_ref[...], preferred_element_type=jnp.float32)
    h = jnp.maximum(h + b1_ref[...], 0.0)
    h = jnp.maximum(jnp.dot(h, w2_ref[...], preferred_element_type=jnp.float32)
                    + b2_ref[...], 0.0)
    h = jnp.maximum(jnp.dot(h, w3_ref[...], preferred_element_type=jnp.float32)
                    + b3_ref[...], 0.0)
    mlp = jnp.sum(h * wo_ref[...], axis=1)
    logit = linear + fm + mlp + cb_ref[0, 0]
    out_ref[...] = jax.nn.sigmoid(logit)[None, None, :]


def _tc_forward(h, eA, eB, eC, w1a, w1b, w1c, b1, w2, b2, w3, b3, wo, cb):
    grid = _BH // _BLK
    full = lambda i: (0, 0)
    out = pl.pallas_call(
        _tc_body,
        grid=(grid,),
        in_specs=[
            pl.BlockSpec((2, _BLK, 128), lambda i: (0, h * (_BH // _BLK) + i, 0)),
            pl.BlockSpec((1, _BLK, 128), lambda i: (0, h * (_BH // _BLK) + i, 0)),
            pl.BlockSpec((1, _BLK, 128), lambda i: (0, i, 0)),
            pl.BlockSpec(w1a.shape, full),
            pl.BlockSpec(w1b.shape, full),
            pl.BlockSpec(w1c.shape, full),
            pl.BlockSpec(b1.shape, full),
            pl.BlockSpec(w2.shape, full),
            pl.BlockSpec(b2.shape, full),
            pl.BlockSpec(w3.shape, full),
            pl.BlockSpec(b3.shape, full),
            pl.BlockSpec(wo.shape, full),
            pl.BlockSpec(cb.shape, full),
        ],
        out_specs=pl.BlockSpec((1, 1, _BLK), lambda i: (i, 0, 0)),
        out_shape=jax.ShapeDtypeStruct((grid, 1, _BLK), jnp.float32),
    )(eA, eB, eC, w1a, w1b, w1c, b1, w2, b2, w3, b3, wo, cb)
    return out.reshape(_BH)


def kernel(x, additional, column, table, bias, W1, b1, g1, be1, W2, b2, g2,
           be2, W3, b3, g3, be3, Wo, bo):
    xt = x.T
    x_sel = xt[jnp.asarray(_FIELD_COLS)]
    idx = (x_sel + jnp.asarray(_OFFSETS)[:, None]).astype(jnp.int32)
    idxA = idx[:10].reshape(_NW, -1, _IDX_W)
    idxB = (idx[10:16] - _B_BASE).reshape(_NW, -1, _IDX_W)

    table_t = table.T
    tlA = _relayout_table(table_t, 0, _A_BLKS)
    tlB = _relayout_table(table_t, _B_LO, _B_BLKS)
    tlC = _relayout_table(table_t, _C_LO, _C_BLKS)

    outA = _sc_gather_scatter(tlA, idxA, jnp.asarray(_GA), 2, BATCH)
    eA = outA.reshape(2, BATCH, 128)
    outB = _sc_gather_scatter(tlB, idxB, jnp.asarray(_GB), 1, BATCH)
    eB = outB.reshape(1, BATCH, 128)
    eCs = []
    for h in range(2):
        idxC = (idx[16:, h * _BH:(h + 1) * _BH] - _C_BASE
                ).reshape(_NW, -1, _IDX_W)
        outC = _sc_gather_scatter(tlC, idxC, jnp.asarray(_GC), 1, _BH)
        eCs.append(outC.reshape(1, _BH, 128))

    inv = np.float32(1.0 / np.sqrt(1.0 + 1e-5))
    s1, s2, s3 = inv * g1, inv * g2, inv * g3
    w1 = W1 * s1[None, :]
    w1a = w1[0:128]
    w1b = w1[128:256]
    w1c = jnp.pad(w1[256:304], ((0, 80), (0, 0)))
    w2 = W2 * s2[None, :]
    w3 = W3 * s3[None, :]
    b1e = (b1 * s1 + be1)[None, :]
    b2e = (b2 * s2 + be2)[None, :]
    b3e = (b3 * s3 + be3)[None, :]
    wo = Wo.reshape(1, -1)
    cb = (bias[0] + bo[0]).reshape(1, 1)

    outs = [_tc_forward(h, eA, eB, eCs[h], w1a, w1b, w1c, b1e, w2, b2e, w3,
                        b3e, wo, cb)
            for h in range(2)]
    return jnp.concatenate(outs)
